# baseline (speedup 1.0000x reference)
import dataclasses
import functools

import jax
import jax.numpy as jnp
from jax import lax
from jax.experimental import pallas as pl
from jax.experimental.pallas import tpu as pltpu
from jax.experimental.pallas import tpu_sc as plsc

N = 10000
E = 160000
D = 256
DE = 16

NUM_SC = 2
NUM_TILES = 16
NW = NUM_SC * NUM_TILES
LANES = 16

N_PAD = 10240
RPT = N_PAD // NW
TRASH = RPT
ACC_ROWS = RPT + 8

E_PAD = 163840
SCH = 2048
NSCAN = E_PAD // SCH
B = 32
CAP = SCH + 4 * LANES

_SC_PARAMS = pltpu.CompilerParams()
if "needs_layout_passes" in pltpu.CompilerParams.__dataclass_fields__:
    _SC_PARAMS = dataclasses.replace(_SC_PARAMS, needs_layout_passes=False)


BLK_E = 2048


def _eproj_body(ef_ref, we_ref, be_ref, out_ref):
    out_ref[...] = (
        jnp.dot(ef_ref[...], we_ref[...], preferred_element_type=jnp.float32)
        + be_ref[...]
    )


def _eproj(ef_pad, W_e, b_e):
    return pl.pallas_call(
        _eproj_body,
        grid=(E_PAD // BLK_E,),
        in_specs=[
            pl.BlockSpec((BLK_E, DE), lambda i: (i, 0)),
            pl.BlockSpec((DE, D), lambda i: (0, 0)),
            pl.BlockSpec((1, D), lambda i: (0, 0)),
        ],
        out_specs=pl.BlockSpec((BLK_E, D), lambda i: (i, 0)),
        out_shape=jax.ShapeDtypeStruct((E_PAD, D), jnp.float32),
    )(ef_pad, W_e, b_e.reshape(1, D))


_SC_MESH = plsc.VectorSubcoreMesh(core_axis_name="c", subcore_axis_name="s")


@functools.partial(
    pl.kernel,
    out_type=jax.ShapeDtypeStruct((N_PAD, D), jnp.float32),
    mesh=_SC_MESH,
    compiler_params=_SC_PARAMS,
    scratch_types=[
        pltpu.VMEM((SCH,), jnp.int32),
        pltpu.VMEM((SCH,), jnp.int32),
        pltpu.VMEM((CAP,), jnp.int32),
        pltpu.VMEM((CAP,), jnp.int32),
        pltpu.VMEM((CAP,), jnp.int32),
        pltpu.VMEM((B, D), jnp.float32),
        pltpu.VMEM((B, D), jnp.float32),
        pltpu.VMEM((B, D), jnp.float32),
        pltpu.VMEM((B, D), jnp.float32),
        pltpu.VMEM((ACC_ROWS, D), jnp.float32),
        pltpu.SemaphoreType.DMA,
        pltpu.SemaphoreType.DMA,
        pltpu.SemaphoreType.DMA,
    ],
)
def _sc_edge(nf_hbm, ep_hbm, s_hbm, r_hbm, agg_hbm,
             rv, sv, idb, sdb, lrb, gbufA, ebufA, gbufB, ebufB, acc,
             semA, semB, semS):
    c = lax.axis_index("c")
    s = lax.axis_index("s")
    w = c * NUM_TILES + s
    lo = w * RPT
    lane = lax.broadcasted_iota(jnp.int32, (LANES,), 0)

    @pl.loop(0, ACC_ROWS)
    def _(i):
        for t in range(D // LANES):
            acc[i, pl.ds(t * LANES, LANES)] = jnp.zeros((LANES,), jnp.float32)

    def issue_scan(sc1):
        pltpu.async_copy(r_hbm.at[pl.ds(sc1 * SCH, SCH)], rv, semS)
        pltpu.async_copy(s_hbm.at[pl.ds(sc1 * SCH, SCH)], sv, semS)

    def wait_scan():
        pltpu.make_async_copy(r_hbm.at[pl.ds(0, SCH)], rv, semS).wait()
        pltpu.make_async_copy(s_hbm.at[pl.ds(0, SCH)], sv, semS).wait()

    def issue_batch(j, gb, eb, sem):
        bsl = pl.ds(j * B, B)
        pltpu.async_copy(nf_hbm.at[sdb.at[bsl]], gb, sem)
        pltpu.async_copy(ep_hbm.at[idb.at[bsl]], eb, sem)

    def wait_batch(gb, eb, sem):
        pltpu.make_async_copy(nf_hbm.at[sdb.at[pl.ds(0, B)]], gb, sem).wait()
        pltpu.make_async_copy(ep_hbm.at[idb.at[pl.ds(0, B)]], eb, sem).wait()

    def acc_batch(j, gb, eb):
        @pl.loop(0, B // LANES)
        def _(g):
            lr16 = lrb[pl.ds(j * B + g * LANES, LANES)]
            for jj in range(LANES):
                r = jnp.sum(jnp.where(lane == jj, lr16, 0))
                row = g * LANES + jj
                for t in range(D // LANES):
                    csl = pl.ds(t * LANES, LANES)
                    msg = jnp.maximum(gb[row, csl] + eb[row, csl], 0.0)
                    acc[r, csl] = acc[r, csl] + msg

    issue_scan(0)

    @pl.loop(0, NSCAN)
    def _(sc):
        wait_scan()

        def scan_body(g, cnt):
            sl = pl.ds(g * LANES, LANES)
            r16 = rv[sl]
            lr16 = r16 - lo
            m = (lr16 >= 0) & (lr16 < RPT)
            ids16 = sc * SCH + g * LANES + lane
            plsc.store_compressed(idb.at[pl.ds(cnt, LANES)], ids16, mask=m)
            plsc.store_compressed(sdb.at[pl.ds(cnt, LANES)], sv[sl], mask=m)
            plsc.store_compressed(lrb.at[pl.ds(cnt, LANES)], lr16, mask=m)
            return cnt + jnp.max(plsc.all_reduce_population_count(m))

        cnt = pl.loop(0, SCH // LANES, init_carry=jnp.int32(0))(scan_body)

        @pl.when(sc + 1 < NSCAN)
        def _():
            issue_scan(sc + 1)

        for p in range(2 * B // LANES):
            off = pl.ds(cnt + p * LANES, LANES)
            idb[off] = jnp.zeros((LANES,), jnp.int32)
            sdb[off] = jnp.zeros((LANES,), jnp.int32)
            lrb[off] = jnp.full((LANES,), TRASH, jnp.int32)

        npairs = (cnt + 2 * B - 1) // (2 * B)

        @pl.when(npairs > 0)
        def _():
            issue_batch(0, gbufA, ebufA, semA)
            issue_batch(1, gbufB, ebufB, semB)

        @pl.loop(0, npairs)
        def _(i):
            wait_batch(gbufA, ebufA, semA)
            acc_batch(2 * i, gbufA, ebufA)

            @pl.when(i + 1 < npairs)
            def _():
                issue_batch(2 * i + 2, gbufA, ebufA, semA)

            wait_batch(gbufB, ebufB, semB)
            acc_batch(2 * i + 1, gbufB, ebufB)

            @pl.when(i + 1 < npairs)
            def _():
                issue_batch(2 * i + 3, gbufB, ebufB, semB)

    pltpu.sync_copy(acc.at[pl.ds(0, RPT)], agg_hbm.at[pl.ds(w * RPT, RPT)])


BLK_N = 1000


def _mlp_body(nf_ref, agg_ref, g_ref, w1_ref, b1_ref, w2_ref,
              b2_ref, wg_ref, bg_ref, eps_ref, out_ref):
    h = (1.0 + eps_ref[...]) * nf_ref[...] + agg_ref[...]
    h = jnp.maximum(
        jnp.dot(h, w1_ref[...], preferred_element_type=jnp.float32) + b1_ref[...],
        0.0,
    )
    h = jnp.dot(h, w2_ref[...], preferred_element_type=jnp.float32) + b2_ref[...]
    g = jnp.dot(g_ref[...], wg_ref[...], preferred_element_type=jnp.float32) + bg_ref[...]
    out_ref[...] = jnp.maximum(h + g, 0.0)


def _mlp(node_features, agg, global_features, W1, b1, W2, b2, W_g, b_g,
         epsilon):
    full = lambda i: (0, 0)
    return pl.pallas_call(
        _mlp_body,
        grid=(N // BLK_N,),
        in_specs=[
            pl.BlockSpec((BLK_N, D), lambda i: (i, 0)),
            pl.BlockSpec((BLK_N, D), lambda i: (i, 0)),
            pl.BlockSpec((1, D), full),
            pl.BlockSpec((D, D), full),
            pl.BlockSpec((1, D), full),
            pl.BlockSpec((D, D), full),
            pl.BlockSpec((1, D), full),
            pl.BlockSpec((D, D), full),
            pl.BlockSpec((1, D), full),
            pl.BlockSpec((1, 1), full),
        ],
        out_specs=pl.BlockSpec((BLK_N, D), lambda i: (i, 0)),
        out_shape=jax.ShapeDtypeStruct((N, D), jnp.float32),
    )(node_features, agg, global_features, W1, b1.reshape(1, D),
      W2, b2.reshape(1, D), W_g, b_g.reshape(1, D), epsilon)


def kernel(node_features, edge_features, global_features, senders, receivers,
           W_e, b_e, W1, b1, W2, b2, W_g, b_g, epsilon):
    ef_pad = jnp.pad(edge_features, ((0, E_PAD - E), (0, 0)))
    s_pad = jnp.pad(senders, (0, E_PAD - E))
    r_pad = jnp.pad(receivers, (0, E_PAD - E), constant_values=-1)

    eproj = _eproj(ef_pad, W_e, b_e)
    agg = _sc_edge(node_features, eproj, s_pad, r_pad)
    return _mlp(node_features, agg, global_features,
                W1, b1, W2, b2, W_g, b_g, epsilon)


# speedup vs baseline: 1.9374x; 1.9374x over previous
import dataclasses
import functools

import jax
import jax.numpy as jnp
from jax import lax
from jax.experimental import pallas as pl
from jax.experimental.pallas import tpu as pltpu
from jax.experimental.pallas import tpu_sc as plsc

N = 10000
E = 160000
D = 256
DE = 16

NUM_SC = 2
NUM_TILES = 16
NW = NUM_SC * NUM_TILES
LANES = 16

N_PAD = 10240
RPT = N_PAD // NW
TRASH = RPT
ACC_ROWS = RPT + 8

E_PAD = 163840
SCH = 4096
NSCAN = E_PAD // SCH
B = 32
CAP = SCH + 2 * LANES

_SC_PARAMS = pltpu.CompilerParams()
if "needs_layout_passes" in pltpu.CompilerParams.__dataclass_fields__:
    _SC_PARAMS = dataclasses.replace(_SC_PARAMS, needs_layout_passes=False)


BLK_E = 2048


def _eproj_body(ef_ref, we_ref, be_ref, out_ref):
    out_ref[...] = (
        jnp.dot(ef_ref[...], we_ref[...], preferred_element_type=jnp.float32)
        + be_ref[...]
    )


def _eproj(ef_pad, W_e, b_e):
    return pl.pallas_call(
        _eproj_body,
        grid=(E_PAD // BLK_E,),
        in_specs=[
            pl.BlockSpec((BLK_E, DE), lambda i: (i, 0)),
            pl.BlockSpec((DE, D), lambda i: (0, 0)),
            pl.BlockSpec((1, D), lambda i: (0, 0)),
        ],
        out_specs=pl.BlockSpec((BLK_E, D), lambda i: (i, 0)),
        out_shape=jax.ShapeDtypeStruct((E_PAD, D), jnp.float32),
    )(ef_pad, W_e, b_e.reshape(1, D))


_SC_MESH = plsc.VectorSubcoreMesh(core_axis_name="c", subcore_axis_name="s")


@functools.partial(
    pl.kernel,
    out_type=jax.ShapeDtypeStruct((N_PAD, D), jnp.float32),
    mesh=_SC_MESH,
    compiler_params=_SC_PARAMS,
    scratch_types=[
        pltpu.VMEM((SCH,), jnp.int32),
        pltpu.VMEM((SCH,), jnp.int32),
        pltpu.VMEM((CAP,), jnp.int32),
        pltpu.VMEM((CAP,), jnp.int32),
        pltpu.VMEM((CAP,), jnp.int32),
        pltpu.VMEM((B, D), jnp.float32),
        pltpu.VMEM((B, D), jnp.float32),
        pltpu.VMEM((ACC_ROWS, D), jnp.float32),
        pltpu.SemaphoreType.DMA,
        pltpu.SemaphoreType.DMA,
    ],
)
def _sc_edge(nf_hbm, ep_hbm, s_hbm, r_hbm, agg_hbm,
             rv, sv, idb, sdb, lrb, gbuf, ebuf, acc, sem, sem2):
    c = lax.axis_index("c")
    s = lax.axis_index("s")
    w = c * NUM_TILES + s
    lo = w * RPT
    lane = lax.broadcasted_iota(jnp.int32, (LANES,), 0)

    @pl.loop(0, ACC_ROWS)
    def _(i):
        for t in range(D // LANES):
            acc[i, pl.ds(t * LANES, LANES)] = jnp.zeros((LANES,), jnp.float32)

    @pl.loop(0, NSCAN)
    def _(sc):
        pltpu.sync_copy(r_hbm.at[pl.ds(sc * SCH, SCH)], rv)
        pltpu.sync_copy(s_hbm.at[pl.ds(sc * SCH, SCH)], sv)

        def scan_body(g, cnt):
            sl = pl.ds(g * LANES, LANES)
            r16 = rv[sl]
            lr16 = r16 - lo
            m = (lr16 >= 0) & (lr16 < RPT)
            ids16 = sc * SCH + g * LANES + lane
            plsc.store_compressed(idb.at[pl.ds(cnt, LANES)], ids16, mask=m)
            plsc.store_compressed(sdb.at[pl.ds(cnt, LANES)], sv[sl], mask=m)
            plsc.store_compressed(lrb.at[pl.ds(cnt, LANES)], lr16, mask=m)
            return cnt + jnp.max(plsc.all_reduce_population_count(m))

        cnt = pl.loop(0, SCH // LANES, init_carry=jnp.int32(0))(scan_body)

        for p in range(B // LANES):
            off = pl.ds(cnt + p * LANES, LANES)
            idb[off] = jnp.zeros((LANES,), jnp.int32)
            sdb[off] = jnp.zeros((LANES,), jnp.int32)
            lrb[off] = jnp.full((LANES,), TRASH, jnp.int32)

        nbat = (cnt + B - 1) // B

        @pl.loop(0, nbat)
        def _(b):
            bsl = pl.ds(b * B, B)
            d1 = pltpu.async_copy(nf_hbm.at[sdb.at[bsl]], gbuf, sem)
            d2 = pltpu.async_copy(ep_hbm.at[idb.at[bsl]], ebuf, sem2)
            d1.wait()
            d2.wait()

            @pl.loop(0, B // LANES)
            def _(g):
                lr16 = lrb[pl.ds(b * B + g * LANES, LANES)]
                for j in range(LANES):
                    r = jnp.sum(jnp.where(lane == j, lr16, 0))
                    row = g * LANES + j
                    for t in range(D // LANES):
                        csl = pl.ds(t * LANES, LANES)
                        msg = jnp.maximum(gbuf[row, csl] + ebuf[row, csl], 0.0)
                        acc[r, csl] = acc[r, csl] + msg

    pltpu.sync_copy(acc.at[pl.ds(0, RPT)], agg_hbm.at[pl.ds(w * RPT, RPT)])


BLK_N = 1000


def _mlp_body(nf_ref, agg_ref, g_ref, w1_ref, b1_ref, w2_ref,
              b2_ref, wg_ref, bg_ref, eps_ref, out_ref):
    h = (1.0 + eps_ref[...]) * nf_ref[...] + agg_ref[...]
    h = jnp.maximum(
        jnp.dot(h, w1_ref[...], preferred_element_type=jnp.float32) + b1_ref[...],
        0.0,
    )
    h = jnp.dot(h, w2_ref[...], preferred_element_type=jnp.float32) + b2_ref[...]
    g = jnp.dot(g_ref[...], wg_ref[...], preferred_element_type=jnp.float32) + bg_ref[...]
    out_ref[...] = jnp.maximum(h + g, 0.0)


def _mlp(node_features, agg, global_features, W1, b1, W2, b2, W_g, b_g,
         epsilon):
    full = lambda i: (0, 0)
    return pl.pallas_call(
        _mlp_body,
        grid=(N // BLK_N,),
        in_specs=[
            pl.BlockSpec((BLK_N, D), lambda i: (i, 0)),
            pl.BlockSpec((BLK_N, D), lambda i: (i, 0)),
            pl.BlockSpec((1, D), full),
            pl.BlockSpec((D, D), full),
            pl.BlockSpec((1, D), full),
            pl.BlockSpec((D, D), full),
            pl.BlockSpec((1, D), full),
            pl.BlockSpec((D, D), full),
            pl.BlockSpec((1, D), full),
            pl.BlockSpec((1, 1), full),
        ],
        out_specs=pl.BlockSpec((BLK_N, D), lambda i: (i, 0)),
        out_shape=jax.ShapeDtypeStruct((N, D), jnp.float32),
    )(node_features, agg, global_features, W1, b1.reshape(1, D),
      W2, b2.reshape(1, D), W_g, b_g.reshape(1, D), epsilon)


def kernel(node_features, edge_features, global_features, senders, receivers,
           W_e, b_e, W1, b1, W2, b2, W_g, b_g, epsilon):
    ef_pad = jnp.pad(edge_features, ((0, E_PAD - E), (0, 0)))
    s_pad = jnp.pad(senders, (0, E_PAD - E))
    r_pad = jnp.pad(receivers, (0, E_PAD - E), constant_values=-1)

    eproj = _eproj(ef_pad, W_e, b_e)
    agg = _sc_edge(node_features, eproj, s_pad, r_pad)
    return _mlp(node_features, agg, global_features,
                W1, b1, W2, b2, W_g, b_g, epsilon)
